# baseline (speedup 1.0000x reference)
import functools

import jax
import jax.numpy as jnp
from jax import lax
from jax.experimental import pallas as pl
from jax.experimental.pallas import tpu as pltpu
from jax.experimental.pallas import tpu_sc as plsc

Q = 1024
D = 32
K = 100000
TOPK = 8
LANES = 128
BK = 512
KPAD = 100352
NSTEPS = KPAD // BK
SUB = BK // LANES
NEG = float("-inf")
BIGI = 2**30

NC = 2
NS = 16
NW = NC * NS
ROWS_PER_W = Q // NW
NGRP = LANES // 16


def _topk_body(q_ref, k_ref, rvo_ref, rio_ref, rv_ref, ri_ref):
    j = pl.program_id(0)

    @pl.when(j == 0)
    def _init():
        rv_ref[...] = jnp.full((TOPK, Q, LANES), NEG, jnp.float32)
        ri_ref[...] = jnp.zeros((TOPK, Q, LANES), jnp.int32)

    sims = jax.lax.dot_general(
        q_ref[...], k_ref[...],
        (((1,), (1,)), ((), ())),
        preferred_element_type=jnp.float32,
        precision=jax.lax.Precision.DEFAULT,
    )
    base = j * BK
    colid = base + jax.lax.broadcasted_iota(jnp.int32, (Q, BK), 1)
    sims = jnp.where(colid < K, sims, NEG)

    for t in range(SUB):
        nv = sims[:, t * LANES:(t + 1) * LANES]
        ni = colid[:, t * LANES:(t + 1) * LANES]
        for i in range(TOPK):
            rv_i = rv_ref[i]
            ri_i = ri_ref[i]
            cond = nv > rv_i
            rv_ref[i] = jnp.where(cond, nv, rv_i)
            ri_ref[i] = jnp.where(cond, ni, ri_i)
            nv = jnp.where(cond, rv_i, nv)
            ni = jnp.where(cond, ri_i, ni)

    @pl.when(j == NSTEPS - 1)
    def _emit():
        for i in range(TOPK):
            rvo_ref[:, i, :] = rv_ref[i]
            rio_ref[:, i, :] = ri_ref[i]


@jax.jit
def _tc_stage(qn, kn_padded):
    return pl.pallas_call(
        _topk_body,
        grid=(NSTEPS,),
        in_specs=[
            pl.BlockSpec((Q, D), lambda j: (0, 0)),
            pl.BlockSpec((BK, D), lambda j: (j, 0)),
        ],
        out_specs=[
            pl.BlockSpec((Q, TOPK, LANES), lambda j: (0, 0, 0)),
            pl.BlockSpec((Q, TOPK, LANES), lambda j: (0, 0, 0)),
        ],
        out_shape=[
            jax.ShapeDtypeStruct((Q, TOPK, LANES), jnp.float32),
            jax.ShapeDtypeStruct((Q, TOPK, LANES), jnp.int32),
        ],
        scratch_shapes=[
            pltpu.VMEM((TOPK, Q, LANES), jnp.float32),
            pltpu.VMEM((TOPK, Q, LANES), jnp.int32),
        ],
        compiler_params=pltpu.CompilerParams(
            dimension_semantics=("arbitrary",),
        ),
    )(qn, kn_padded)


def _scalar_max(v):
    m = v[0]
    for l in range(1, 16):
        m = jnp.maximum(m, v[l])
    return m


def _scalar_min_i32(v):
    m = v[0]
    for l in range(1, 16):
        m = jnp.minimum(m, v[l])
    return m


def _merge_body(rv_hbm, ri_hbm, vals_hbm, idx_hbm, v_v, i_v, vo_v, io_v):
    wid = lax.axis_index("s") * NC + lax.axis_index("c")
    r0 = wid * ROWS_PER_W
    iota16 = lax.iota(jnp.int32, 16)

    def row_body(rr, _):
        r = r0 + rr
        pltpu.sync_copy(rv_hbm.at[r], v_v)
        pltpu.sync_copy(ri_hbm.at[r], i_v)

        fr_v, fr_i, ptr = [], [], []
        for g in range(NGRP):
            fr_v.append(v_v[0, pl.ds(g * 16, 16)])
            fr_i.append(i_v[0, pl.ds(g * 16, 16)])
            ptr.append(jnp.zeros((16,), jnp.int32))
        vout = jnp.full((16,), 0.0, jnp.float32)
        iout = jnp.zeros((16,), jnp.int32)

        for p in range(TOPK):
            m16 = fr_v[0]
            for g in range(1, NGRP):
                m16 = jnp.maximum(m16, fr_v[g])
            mv = jnp.full((16,), _scalar_max(m16), jnp.float32)
            big16 = jnp.full((16,), BIGI, jnp.int32)
            wi16 = big16
            for g in range(NGRP):
                wi16 = jnp.minimum(
                    wi16, jnp.where(fr_v[g] == mv, fr_i[g], big16))
            widxv = jnp.full((16,), _scalar_min_i32(wi16), jnp.int32)
            vout = jnp.where(iota16 == p, mv, vout)
            iout = jnp.where(iota16 == p, widxv, iout)
            one16 = jnp.full((16,), 1, jnp.int32)
            zero16 = jnp.full((16,), 0, jnp.int32)
            neg16 = jnp.full((16,), NEG, jnp.float32)
            for g in range(NGRP):
                win = (jnp.where(fr_v[g] == mv, one16, zero16)
                       * jnp.where(fr_i[g] == widxv, one16, zero16))
                ptr[g] = ptr[g] + win
                nfv = v_v[0, pl.ds(g * 16, 16)]
                nfi = i_v[0, pl.ds(g * 16, 16)]
                for l in range(1, TOPK):
                    lv = jnp.full((16,), l, jnp.int32)
                    sel = ptr[g] == lv
                    nfv = jnp.where(sel, v_v[l, pl.ds(g * 16, 16)], nfv)
                    nfi = jnp.where(sel, i_v[l, pl.ds(g * 16, 16)], nfi)
                k16 = jnp.full((16,), TOPK - 1, jnp.int32)
                fr_v[g] = jnp.where(ptr[g] > k16, neg16, nfv)
                fr_i[g] = nfi

        vo_v[pl.ds(0, 16)] = vout
        io_v[pl.ds(0, 16)] = iout
        pltpu.sync_copy(vo_v.at[pl.ds(0, TOPK)], vals_hbm.at[pl.ds(r * TOPK, TOPK)])
        pltpu.sync_copy(io_v.at[pl.ds(0, TOPK)], idx_hbm.at[pl.ds(r * TOPK, TOPK)])
        return 0

    lax.fori_loop(0, ROWS_PER_W, row_body, 0)


@jax.jit
def _sc_stage(rv, ri):
    merge = functools.partial(
        pl.kernel,
        mesh=plsc.VectorSubcoreMesh(core_axis_name="c", subcore_axis_name="s"),
        out_type=[
            jax.ShapeDtypeStruct((Q * TOPK,), jnp.float32),
            jax.ShapeDtypeStruct((Q * TOPK,), jnp.int32),
        ],
        scratch_types=[
            pltpu.VMEM((TOPK, LANES), jnp.float32),
            pltpu.VMEM((TOPK, LANES), jnp.int32),
            pltpu.VMEM((128,), jnp.float32),
            pltpu.VMEM((128,), jnp.int32),
        ],
    )(_merge_body)
    return merge(rv, ri)


def kernel(queries, keys):
    qn = queries / (jnp.linalg.norm(queries, axis=-1, keepdims=True) + 1e-9)
    kn = keys / (jnp.linalg.norm(keys, axis=-1, keepdims=True) + 1e-9)
    kn_padded = jnp.pad(kn, ((0, KPAD - K), (0, 0)))
    rv, ri = _tc_stage(qn, kn_padded)
    vals_flat, idx_flat = _sc_stage(rv, ri)
    return vals_flat.reshape(Q, TOPK), idx_flat.reshape(Q, TOPK)


# speedup vs baseline: 1.0241x; 1.0241x over previous
import functools

import jax
import jax.numpy as jnp
from jax import lax
from jax.experimental import pallas as pl
from jax.experimental.pallas import tpu as pltpu
from jax.experimental.pallas import tpu_sc as plsc

Q = 1024
D = 32
K = 100000
TOPK = 8
LANES = 128
BK = 512
KPAD = 100352
NSTEPS = KPAD // BK
SUB = BK // LANES
NEG = float("-inf")
BIGI = 2**30

NC = 2
NS = 16
NW = NC * NS
ROWS_PER_W = Q // NW
NGRP = LANES // 16


def _topk_body(q_ref, k_ref, rvo_ref, rio_ref, rv_ref, ri_ref):
    j = pl.program_id(0)

    @pl.when(j == 0)
    def _init():
        rv_ref[...] = jnp.full((TOPK, Q, LANES), NEG, jnp.float32)
        ri_ref[...] = jnp.zeros((TOPK, Q, LANES), jnp.int32)

    sims = jax.lax.dot_general(
        q_ref[...], k_ref[...],
        (((1,), (1,)), ((), ())),
        preferred_element_type=jnp.float32,
        precision=jax.lax.Precision.DEFAULT,
    )
    base = j * BK
    colid = base + jax.lax.broadcasted_iota(jnp.int32, (Q, BK), 1)
    sims = jnp.where(colid < K, sims, NEG)

    nvs = [sims[:, t * LANES:(t + 1) * LANES] for t in range(SUB)]
    nis = [colid[:, t * LANES:(t + 1) * LANES] for t in range(SUB)]
    for i in range(TOPK):
        sv = rv_ref[i]
        si = ri_ref[i]
        for t in range(SUB):
            cond = nvs[t] > sv
            sv_n = jnp.where(cond, nvs[t], sv)
            si_n = jnp.where(cond, nis[t], si)
            nvs[t] = jnp.minimum(nvs[t], sv)
            nis[t] = jnp.where(cond, si, nis[t])
            sv = sv_n
            si = si_n
        rv_ref[i] = sv
        ri_ref[i] = si

    @pl.when(j == NSTEPS - 1)
    def _emit():
        for i in range(TOPK):
            rvo_ref[:, i, :] = rv_ref[i]
            rio_ref[:, i, :] = ri_ref[i]


@jax.jit
def _tc_stage(qn, kn_padded):
    return pl.pallas_call(
        _topk_body,
        grid=(NSTEPS,),
        in_specs=[
            pl.BlockSpec((Q, D), lambda j: (0, 0)),
            pl.BlockSpec((BK, D), lambda j: (j, 0)),
        ],
        out_specs=[
            pl.BlockSpec((Q, TOPK, LANES), lambda j: (0, 0, 0)),
            pl.BlockSpec((Q, TOPK, LANES), lambda j: (0, 0, 0)),
        ],
        out_shape=[
            jax.ShapeDtypeStruct((Q, TOPK, LANES), jnp.float32),
            jax.ShapeDtypeStruct((Q, TOPK, LANES), jnp.int32),
        ],
        scratch_shapes=[
            pltpu.VMEM((TOPK, Q, LANES), jnp.float32),
            pltpu.VMEM((TOPK, Q, LANES), jnp.int32),
        ],
        compiler_params=pltpu.CompilerParams(
            dimension_semantics=("arbitrary",),
        ),
    )(qn, kn_padded)


def _scalar_max(v):
    m = v[0]
    for l in range(1, 16):
        m = jnp.maximum(m, v[l])
    return m


def _scalar_min_i32(v):
    m = v[0]
    for l in range(1, 16):
        m = jnp.minimum(m, v[l])
    return m


def _merge_body(rv_hbm, ri_hbm, vals_hbm, idx_hbm, v_v, i_v, vo_v, io_v):
    wid = lax.axis_index("s") * NC + lax.axis_index("c")
    r0 = wid * ROWS_PER_W
    iota16 = lax.iota(jnp.int32, 16)

    def row_body(rr, _):
        r = r0 + rr
        pltpu.sync_copy(rv_hbm.at[r], v_v)
        pltpu.sync_copy(ri_hbm.at[r], i_v)

        fr_v, fr_i, ptr = [], [], []
        for g in range(NGRP):
            fr_v.append(v_v[0, pl.ds(g * 16, 16)])
            fr_i.append(i_v[0, pl.ds(g * 16, 16)])
            ptr.append(jnp.zeros((16,), jnp.int32))
        vout = jnp.full((16,), 0.0, jnp.float32)
        iout = jnp.zeros((16,), jnp.int32)

        for p in range(TOPK):
            m16 = fr_v[0]
            for g in range(1, NGRP):
                m16 = jnp.maximum(m16, fr_v[g])
            mv = jnp.full((16,), _scalar_max(m16), jnp.float32)
            big16 = jnp.full((16,), BIGI, jnp.int32)
            wi16 = big16
            for g in range(NGRP):
                wi16 = jnp.minimum(
                    wi16, jnp.where(fr_v[g] == mv, fr_i[g], big16))
            widxv = jnp.full((16,), _scalar_min_i32(wi16), jnp.int32)
            vout = jnp.where(iota16 == p, mv, vout)
            iout = jnp.where(iota16 == p, widxv, iout)
            one16 = jnp.full((16,), 1, jnp.int32)
            zero16 = jnp.full((16,), 0, jnp.int32)
            neg16 = jnp.full((16,), NEG, jnp.float32)
            for g in range(NGRP):
                win = (jnp.where(fr_v[g] == mv, one16, zero16)
                       * jnp.where(fr_i[g] == widxv, one16, zero16))
                ptr[g] = ptr[g] + win
                nfv = v_v[0, pl.ds(g * 16, 16)]
                nfi = i_v[0, pl.ds(g * 16, 16)]
                for l in range(1, TOPK):
                    lv = jnp.full((16,), l, jnp.int32)
                    sel = ptr[g] == lv
                    nfv = jnp.where(sel, v_v[l, pl.ds(g * 16, 16)], nfv)
                    nfi = jnp.where(sel, i_v[l, pl.ds(g * 16, 16)], nfi)
                k16 = jnp.full((16,), TOPK - 1, jnp.int32)
                fr_v[g] = jnp.where(ptr[g] > k16, neg16, nfv)
                fr_i[g] = nfi

        vo_v[pl.ds(0, 16)] = vout
        io_v[pl.ds(0, 16)] = iout
        pltpu.sync_copy(vo_v.at[pl.ds(0, TOPK)], vals_hbm.at[pl.ds(r * TOPK, TOPK)])
        pltpu.sync_copy(io_v.at[pl.ds(0, TOPK)], idx_hbm.at[pl.ds(r * TOPK, TOPK)])
        return 0

    lax.fori_loop(0, ROWS_PER_W, row_body, 0)


@jax.jit
def _sc_stage(rv, ri):
    merge = functools.partial(
        pl.kernel,
        mesh=plsc.VectorSubcoreMesh(core_axis_name="c", subcore_axis_name="s"),
        out_type=[
            jax.ShapeDtypeStruct((Q * TOPK,), jnp.float32),
            jax.ShapeDtypeStruct((Q * TOPK,), jnp.int32),
        ],
        scratch_types=[
            pltpu.VMEM((TOPK, LANES), jnp.float32),
            pltpu.VMEM((TOPK, LANES), jnp.int32),
            pltpu.VMEM((128,), jnp.float32),
            pltpu.VMEM((128,), jnp.int32),
        ],
    )(_merge_body)
    return merge(rv, ri)


def kernel(queries, keys):
    qn = queries / (jnp.linalg.norm(queries, axis=-1, keepdims=True) + 1e-9)
    kn = keys / (jnp.linalg.norm(keys, axis=-1, keepdims=True) + 1e-9)
    kn_padded = jnp.pad(kn, ((0, KPAD - K), (0, 0)))
    rv, ri = _tc_stage(qn, kn_padded)
    vals_flat, idx_flat = _sc_stage(rv, ri)
    return vals_flat.reshape(Q, TOPK), idx_flat.reshape(Q, TOPK)


# speedup vs baseline: 1.0269x; 1.0027x over previous
import functools

import jax
import jax.numpy as jnp
from jax import lax
from jax.experimental import pallas as pl
from jax.experimental.pallas import tpu as pltpu
from jax.experimental.pallas import tpu_sc as plsc

Q = 1024
D = 32
K = 100000
TOPK = 8
LANES = 128
BK = 512
KPAD = 100352
NSTEPS = KPAD // BK
SUB = BK // LANES
NEG = float("-inf")
BIGI = 2**30

NC = 2
NS = 16
NW = NC * NS
ROWS_PER_W = Q // NW
NGRP = LANES // 16


def _topk_body(q_ref, k_ref, rvo_ref, rio_ref, rv_ref, ri_ref):
    j = pl.program_id(0)

    @pl.when(j == 0)
    def _init():
        rv_ref[...] = jnp.full((TOPK, Q, LANES), NEG, jnp.float32)
        ri_ref[...] = jnp.zeros((TOPK, Q, LANES), jnp.int32)

    sims = jax.lax.dot_general(
        q_ref[...], k_ref[...],
        (((1,), (1,)), ((), ())),
        preferred_element_type=jnp.float32,
        precision=jax.lax.Precision.DEFAULT,
    )
    base = j * BK
    colid = base + jax.lax.broadcasted_iota(jnp.int32, (Q, BK), 1)
    sims = jnp.where(colid < K, sims, NEG)

    nvs = [sims[:, t * LANES:(t + 1) * LANES] for t in range(SUB)]
    nis = [colid[:, t * LANES:(t + 1) * LANES] for t in range(SUB)]
    for i in range(TOPK):
        sv = rv_ref[i]
        si = ri_ref[i]
        for t in range(SUB):
            cond = nvs[t] > sv
            sv_n = jnp.where(cond, nvs[t], sv)
            si_n = jnp.where(cond, nis[t], si)
            nvs[t] = jnp.minimum(nvs[t], sv)
            nis[t] = jnp.where(cond, si, nis[t])
            sv = sv_n
            si = si_n
        rv_ref[i] = sv
        ri_ref[i] = si

    @pl.when(j == NSTEPS - 1)
    def _emit():
        for i in range(TOPK):
            rvo_ref[:, i, :] = rv_ref[i]
            rio_ref[:, i, :] = ri_ref[i]


@jax.jit
def _tc_stage(qn, kn_padded):
    return pl.pallas_call(
        _topk_body,
        grid=(NSTEPS,),
        in_specs=[
            pl.BlockSpec((Q, D), lambda j: (0, 0)),
            pl.BlockSpec((BK, D), lambda j: (j, 0)),
        ],
        out_specs=[
            pl.BlockSpec((Q, TOPK, LANES), lambda j: (0, 0, 0)),
            pl.BlockSpec((Q, TOPK, LANES), lambda j: (0, 0, 0)),
        ],
        out_shape=[
            jax.ShapeDtypeStruct((Q, TOPK, LANES), jnp.float32),
            jax.ShapeDtypeStruct((Q, TOPK, LANES), jnp.int32),
        ],
        scratch_shapes=[
            pltpu.VMEM((TOPK, Q, LANES), jnp.float32),
            pltpu.VMEM((TOPK, Q, LANES), jnp.int32),
        ],
        compiler_params=pltpu.CompilerParams(
            dimension_semantics=("arbitrary",),
        ),
    )(qn, kn_padded)


def _scalar_max(v):
    m = v[0]
    for l in range(1, 16):
        m = jnp.maximum(m, v[l])
    return m


def _scalar_min_i32(v):
    m = v[0]
    for l in range(1, 16):
        m = jnp.minimum(m, v[l])
    return m


def _merge_body(rv_hbm, ri_hbm, vals_hbm, idx_hbm,
                v_v0, i_v0, v_v1, i_v1, vo_v0, io_v0, vo_v1, io_v1,
                sem0, sem1, osem0, osem1):
    wid = lax.axis_index("s") * NC + lax.axis_index("c")
    r0 = wid * ROWS_PER_W
    iota16 = lax.iota(jnp.int32, 16)

    pltpu.async_copy(rv_hbm.at[r0], v_v0, sem0)
    pltpu.async_copy(ri_hbm.at[r0], i_v0, sem0)
    pltpu.async_copy(rv_hbm.at[r0 + 1], v_v1, sem1)
    pltpu.async_copy(ri_hbm.at[r0 + 1], i_v1, sem1)

    def _do_row(r, k, v_v, i_v, vo_v, io_v, sem, osem):
        pltpu.make_async_copy(rv_hbm.at[r], v_v, sem).wait()
        pltpu.make_async_copy(ri_hbm.at[r], i_v, sem).wait()

        fr_v, fr_i, ptr = [], [], []
        for g in range(NGRP):
            fr_v.append(v_v[0, pl.ds(g * 16, 16)])
            fr_i.append(i_v[0, pl.ds(g * 16, 16)])
            ptr.append(jnp.zeros((16,), jnp.int32))
        vout = jnp.full((16,), 0.0, jnp.float32)
        iout = jnp.zeros((16,), jnp.int32)

        for p in range(TOPK):
            m16 = fr_v[0]
            for g in range(1, NGRP):
                m16 = jnp.maximum(m16, fr_v[g])
            mv = jnp.full((16,), _scalar_max(m16), jnp.float32)
            big16 = jnp.full((16,), BIGI, jnp.int32)
            wi16 = big16
            for g in range(NGRP):
                wi16 = jnp.minimum(
                    wi16, jnp.where(fr_v[g] == mv, fr_i[g], big16))
            widxv = jnp.full((16,), _scalar_min_i32(wi16), jnp.int32)
            vout = jnp.where(iota16 == p, mv, vout)
            iout = jnp.where(iota16 == p, widxv, iout)
            one16 = jnp.full((16,), 1, jnp.int32)
            zero16 = jnp.full((16,), 0, jnp.int32)
            neg16 = jnp.full((16,), NEG, jnp.float32)
            for g in range(NGRP):
                win = (jnp.where(fr_v[g] == mv, one16, zero16)
                       * jnp.where(fr_i[g] == widxv, one16, zero16))
                ptr[g] = ptr[g] + win
                nfv = v_v[0, pl.ds(g * 16, 16)]
                nfi = i_v[0, pl.ds(g * 16, 16)]
                for l in range(1, TOPK):
                    lv = jnp.full((16,), l, jnp.int32)
                    sel = ptr[g] == lv
                    nfv = jnp.where(sel, v_v[l, pl.ds(g * 16, 16)], nfv)
                    nfi = jnp.where(sel, i_v[l, pl.ds(g * 16, 16)], nfi)
                k16 = jnp.full((16,), TOPK - 1, jnp.int32)
                fr_v[g] = jnp.where(ptr[g] > k16, neg16, nfv)
                fr_i[g] = nfi

        @pl.when(k > 0)
        def _drain():
            pltpu.make_async_copy(
                vo_v.at[pl.ds(0, TOPK)],
                vals_hbm.at[pl.ds(r * TOPK, TOPK)], osem).wait()
            pltpu.make_async_copy(
                io_v.at[pl.ds(0, TOPK)],
                idx_hbm.at[pl.ds(r * TOPK, TOPK)], osem).wait()
        vo_v[pl.ds(0, 16)] = vout
        io_v[pl.ds(0, 16)] = iout
        pltpu.async_copy(vo_v.at[pl.ds(0, TOPK)],
                         vals_hbm.at[pl.ds(r * TOPK, TOPK)], osem)
        pltpu.async_copy(io_v.at[pl.ds(0, TOPK)],
                         idx_hbm.at[pl.ds(r * TOPK, TOPK)], osem)
        @pl.when(k < ROWS_PER_W // 2 - 1)
        def _prefetch():
            pltpu.async_copy(rv_hbm.at[r + 2], v_v, sem)
            pltpu.async_copy(ri_hbm.at[r + 2], i_v, sem)

    def pair_body(k, _):
        r = r0 + 2 * k
        _do_row(r, k, v_v0, i_v0, vo_v0, io_v0, sem0, osem0)
        _do_row(r + 1, k, v_v1, i_v1, vo_v1, io_v1, sem1, osem1)
        return 0

    lax.fori_loop(0, ROWS_PER_W // 2, pair_body, 0)

    rlast = r0 + ROWS_PER_W - 2
    pltpu.make_async_copy(vo_v0.at[pl.ds(0, TOPK)],
                          vals_hbm.at[pl.ds(rlast * TOPK, TOPK)], osem0).wait()
    pltpu.make_async_copy(io_v0.at[pl.ds(0, TOPK)],
                          idx_hbm.at[pl.ds(rlast * TOPK, TOPK)], osem0).wait()
    pltpu.make_async_copy(vo_v1.at[pl.ds(0, TOPK)],
                          vals_hbm.at[pl.ds((rlast + 1) * TOPK, TOPK)],
                          osem1).wait()
    pltpu.make_async_copy(io_v1.at[pl.ds(0, TOPK)],
                          idx_hbm.at[pl.ds((rlast + 1) * TOPK, TOPK)],
                          osem1).wait()


@jax.jit
def _sc_stage(rv, ri):
    merge = functools.partial(
        pl.kernel,
        mesh=plsc.VectorSubcoreMesh(core_axis_name="c", subcore_axis_name="s"),
        out_type=[
            jax.ShapeDtypeStruct((Q * TOPK,), jnp.float32),
            jax.ShapeDtypeStruct((Q * TOPK,), jnp.int32),
        ],
        scratch_types=[
            pltpu.VMEM((TOPK, LANES), jnp.float32),
            pltpu.VMEM((TOPK, LANES), jnp.int32),
            pltpu.VMEM((TOPK, LANES), jnp.float32),
            pltpu.VMEM((TOPK, LANES), jnp.int32),
            pltpu.VMEM((128,), jnp.float32),
            pltpu.VMEM((128,), jnp.int32),
            pltpu.VMEM((128,), jnp.float32),
            pltpu.VMEM((128,), jnp.int32),
            pltpu.SemaphoreType.DMA,
            pltpu.SemaphoreType.DMA,
            pltpu.SemaphoreType.DMA,
            pltpu.SemaphoreType.DMA,
        ],
    )(_merge_body)
    return merge(rv, ri)


def kernel(queries, keys):
    qn = queries / (jnp.linalg.norm(queries, axis=-1, keepdims=True) + 1e-9)
    kn = keys / (jnp.linalg.norm(keys, axis=-1, keepdims=True) + 1e-9)
    kn_padded = jnp.pad(kn, ((0, KPAD - K), (0, 0)))
    rv, ri = _tc_stage(qn, kn_padded)
    vals_flat, idx_flat = _sc_stage(rv, ri)
    return vals_flat.reshape(Q, TOPK), idx_flat.reshape(Q, TOPK)
